# baseline (speedup 1.0000x reference)
import functools

import jax
import jax.numpy as jnp
import numpy as np
from jax import lax
from jax.experimental import pallas as pl
from jax.experimental.pallas import tpu as pltpu
from jax.experimental.pallas import tpu_sc as plsc

_FEATURE_FIELDS = [100000] * 26
_OFFSETS = np.array((0, *np.cumsum(_FEATURE_FIELDS)[:-1]), dtype=np.int32)
_B = 4096
_F = 26
_D = 64
_BF = _B * _F
_BN_EPS = 1e-3

_NC = 2
_NS = 16
_NW = _NC * _NS
_CHUNK = 128
_CHUNKS_PER_W = _BF // (_NW * _CHUNK)
_N_PER_W = _CHUNK * _CHUNKS_PER_W

_V2 = 1300000
_VL = 162500

_BM = 256
_H = _F * _D


def _sc_gather(q3, ql3, t2, lp):
    mesh = plsc.VectorSubcoreMesh(core_axis_name="c", subcore_axis_name="s")

    @functools.partial(
        pl.kernel,
        mesh=mesh,
        compiler_params=pltpu.CompilerParams(use_tc_tiling_on_sc=False),
        out_type=(
            jax.ShapeDtypeStruct((_BF, 128), jnp.float32),
            jax.ShapeDtypeStruct((_BF, 16), jnp.float32),
        ),
        scratch_types=(
            pltpu.VMEM((_CHUNKS_PER_W, _CHUNK), jnp.int32),
            pltpu.VMEM((_CHUNKS_PER_W, _CHUNK), jnp.int32),
            pltpu.VMEM((_CHUNK, 128), jnp.float32),
            pltpu.VMEM((_CHUNK, 16), jnp.float32),
            pltpu.SemaphoreType.DMA,
            pltpu.SemaphoreType.DMA,
        ),
    )
    def k(q_hbm, ql_hbm, t2_hbm, lp_hbm, out_hbm, lout_hbm, qv, qlv, ebuf,
          lbuf, esem, lsem):
        wid = lax.axis_index("s") * _NC + lax.axis_index("c")
        base = wid * _N_PER_W
        pltpu.sync_copy(q_hbm.at[wid], qv)
        pltpu.sync_copy(ql_hbm.at[wid], qlv)

        def body(j, carry):
            cpe = pltpu.async_copy(t2_hbm.at[qv.at[j]], ebuf, esem)
            cpl = pltpu.async_copy(lp_hbm.at[qlv.at[j]], lbuf, lsem)
            off = base + j * _CHUNK
            cpe.wait()
            pltpu.sync_copy(ebuf, out_hbm.at[pl.ds(off, _CHUNK)])
            cpl.wait()
            pltpu.sync_copy(lbuf, lout_hbm.at[pl.ds(off, _CHUNK)])
            return carry

        lax.fori_loop(0, _CHUNKS_PER_W, body, 0)

    return k(q3, ql3, t2, lp)


def _tc_body(h_ref, par_ref, l16_ref, m16_ref, bias_ref,
             w1_ref, b1_ref, g1_ref, e1_ref,
             w2_ref, b2_ref, g2_ref, e2_ref,
             w3_ref, b3_ref, g3_ref, e3_ref,
             wout_ref, bout_ref, out_ref):
    inv = lax.rsqrt(jnp.float32(1.0 + _BN_EPS))
    p = par_ref[...]
    hs = []
    for f in range(_F):
        lo = h_ref[:, 128 * f:128 * f + _D]
        hi = h_ref[:, 128 * f + _D:128 * (f + 1)]
        sel = p[:, f:f + 1]
        hs.append(lo + sel * (hi - lo))
    h = jnp.concatenate(hs, axis=1)
    s = hs[0]
    for f in range(1, _F):
        s = s + hs[f]
    sq_of_sum = jnp.sum(s * s, axis=1, keepdims=True)
    sum_of_sq = jnp.sum(h * h, axis=1, keepdims=True)
    cross = 0.5 * (sq_of_sum - sum_of_sq)
    lin = jnp.sum(l16_ref[...] * m16_ref[...], axis=1, keepdims=True)
    lin = lin + bias_ref[0, 0]

    def layer(a, w_ref, b_ref, g_ref, e_ref):
        z = jnp.dot(a, w_ref[...], preferred_element_type=jnp.float32)
        scale = g_ref[...] * inv
        shift = b_ref[...] * scale + e_ref[...]
        return jnp.maximum(z * scale + shift, 0.0)

    a = layer(h, w1_ref, b1_ref, g1_ref, e1_ref)
    a = layer(a, w2_ref, b2_ref, g2_ref, e2_ref)
    a = layer(a, w3_ref, b3_ref, g3_ref, e3_ref)
    mlp = jnp.dot(a, wout_ref[...], preferred_element_type=jnp.float32)
    mlp = mlp + bout_ref[0, 0]
    v = lin + cross + mlp
    out_ref[...] = 1.0 / (1.0 + jnp.exp(-v))


def _tc_mlp(h128, par, l16, m16, bias, W1, b1, g1, be1, W2, b2, g2, be2, W3,
            b3, g3, be3, Wout, bout):
    grid = (_B // _BM,)

    def brow(shape):
        return pl.BlockSpec(shape, lambda i: (i, 0))

    def bfull(shape):
        return pl.BlockSpec(shape, lambda i: (0, 0))

    in_specs = [
        brow((_BM, _F * 128)),
        brow((_BM, _F)),
        brow((_BM, _F * 16)),
        brow((_BM, _F * 16)),
        bfull((1, 1)),
        bfull((_H, 400)), bfull((1, 400)), bfull((1, 400)), bfull((1, 400)),
        bfull((400, 400)), bfull((1, 400)), bfull((1, 400)), bfull((1, 400)),
        bfull((400, 400)), bfull((1, 400)), bfull((1, 400)), bfull((1, 400)),
        bfull((400, 1)), bfull((1, 1)),
    ]
    out_specs = pl.BlockSpec((_BM, 1), lambda i: (i, 0))
    return pl.pallas_call(
        _tc_body,
        grid=grid,
        in_specs=in_specs,
        out_specs=out_specs,
        out_shape=jax.ShapeDtypeStruct((_B, 1), jnp.float32),
    )(h128, par, l16, m16, bias.reshape(1, 1),
      W1, b1.reshape(1, 400), g1.reshape(1, 400), be1.reshape(1, 400),
      W2, b2.reshape(1, 400), g2.reshape(1, 400), be2.reshape(1, 400),
      W3, b3.reshape(1, 400), g3.reshape(1, 400), be3.reshape(1, 400),
      Wout, bout.reshape(1, 1))


def kernel(x, linear_w, embed_w, bias, W1, b1, g1, be1, W2, b2, g2, be2, W3,
           b3, g3, be3, Wout, bout):
    xi = x + jnp.asarray(_OFFSETS, dtype=x.dtype)[None, :]
    xif = xi.reshape(-1)
    q3 = (xif >> 1).reshape(_NW, _CHUNKS_PER_W, _CHUNK)
    ql3 = (xif >> 4).reshape(_NW, _CHUNKS_PER_W, _CHUNK)
    t2 = embed_w[:2 * _V2].reshape(_V2, 128)
    lp = linear_w.reshape(-1)[:16 * _VL].reshape(_VL, 16)
    pairs, lin16 = _sc_gather(q3, ql3, t2, lp)
    h128 = pairs.reshape(_B, _F * 128)
    l16 = lin16.reshape(_B, _F * 16)
    par = (xi & 1).astype(jnp.float32)
    m16 = jax.nn.one_hot(xi & 15, 16, dtype=jnp.float32).reshape(_B, _F * 16)
    return _tc_mlp(h128, par, l16, m16, bias, W1, b1, g1, be1, W2, b2, g2,
                   be2, W3, b3, g3, be3, Wout, bout)


# speedup vs baseline: 1.0156x; 1.0156x over previous
import functools

import jax
import jax.numpy as jnp
import numpy as np
from jax import lax
from jax.experimental import pallas as pl
from jax.experimental.pallas import tpu as pltpu
from jax.experimental.pallas import tpu_sc as plsc

_FEATURE_FIELDS = [100000] * 26
_OFFSETS = np.array((0, *np.cumsum(_FEATURE_FIELDS)[:-1]), dtype=np.int32)
_B = 4096
_F = 26
_D = 64
_BF = _B * _F
_BN_EPS = 1e-3

_NC = 2
_NS = 16
_NW = _NC * _NS
_CHUNK = 128
_CHUNKS_PER_W = _BF // (_NW * _CHUNK)
_N_PER_W = _CHUNK * _CHUNKS_PER_W

_V2 = 1300000
_VL = 162500

_BM = 256
_H = _F * _D


def _sc_gather_emb(q3, t2):
    mesh = plsc.VectorSubcoreMesh(core_axis_name="c", subcore_axis_name="s")

    @functools.partial(
        pl.kernel,
        mesh=mesh,
        compiler_params=pltpu.CompilerParams(use_tc_tiling_on_sc=True),
        out_type=jax.ShapeDtypeStruct((_BF, 128), jnp.float32),
        scratch_types=(
            pltpu.VMEM((_CHUNKS_PER_W, _CHUNK), jnp.int32),
            pltpu.VMEM((2, _CHUNK, 128), jnp.float32),
            pltpu.SemaphoreType.DMA,
        ),
    )
    def k(q_hbm, t2_hbm, out_hbm, qv, ebuf, esem):
        wid = lax.axis_index("s") * _NC + lax.axis_index("c")
        base = wid * _N_PER_W
        pltpu.sync_copy(q_hbm.at[wid], qv)
        pltpu.async_copy(t2_hbm.at[qv.at[0]], ebuf.at[0], esem)

        def body(j, carry):
            slot = lax.rem(j, 2)
            nxt = lax.rem(j + 1, 2)

            @pl.when(j + 1 < _CHUNKS_PER_W)
            def _():
                pltpu.async_copy(t2_hbm.at[qv.at[j + 1]], ebuf.at[nxt], esem)

            pltpu.make_async_copy(
                t2_hbm.at[pl.ds(0, _CHUNK)], ebuf.at[slot], esem).wait()
            pltpu.sync_copy(
                ebuf.at[slot], out_hbm.at[pl.ds(base + j * _CHUNK, _CHUNK)])
            return carry

        lax.fori_loop(0, _CHUNKS_PER_W, body, 0)

    return k(q3, t2)


def _sc_gather_lin(ql3, lp):
    mesh = plsc.VectorSubcoreMesh(core_axis_name="c", subcore_axis_name="s")

    @functools.partial(
        pl.kernel,
        mesh=mesh,
        compiler_params=pltpu.CompilerParams(use_tc_tiling_on_sc=False),
        out_type=jax.ShapeDtypeStruct((_BF, 16), jnp.float32),
        scratch_types=(
            pltpu.VMEM((_CHUNKS_PER_W, _CHUNK), jnp.int32),
            pltpu.VMEM((_CHUNK, 16), jnp.float32),
            pltpu.SemaphoreType.DMA,
        ),
    )
    def k(ql_hbm, lp_hbm, lout_hbm, qlv, lbuf, lsem):
        wid = lax.axis_index("s") * _NC + lax.axis_index("c")
        base = wid * _N_PER_W
        pltpu.sync_copy(ql_hbm.at[wid], qlv)

        def body(j, carry):
            pltpu.async_copy(lp_hbm.at[qlv.at[j]], lbuf, lsem).wait()
            pltpu.sync_copy(
                lbuf, lout_hbm.at[pl.ds(base + j * _CHUNK, _CHUNK)])
            return carry

        lax.fori_loop(0, _CHUNKS_PER_W, body, 0)

    return k(ql3, lp)


def _tc_body(h_ref, par_ref, l16_ref, m16_ref, bias_ref,
             w1_ref, b1_ref, g1_ref, e1_ref,
             w2_ref, b2_ref, g2_ref, e2_ref,
             w3_ref, b3_ref, g3_ref, e3_ref,
             wout_ref, bout_ref, out_ref):
    inv = lax.rsqrt(jnp.float32(1.0 + _BN_EPS))
    p = par_ref[...]
    hs = []
    for f in range(_F):
        lo = h_ref[:, 128 * f:128 * f + _D]
        hi = h_ref[:, 128 * f + _D:128 * (f + 1)]
        sel = p[:, f:f + 1]
        hs.append(lo + sel * (hi - lo))
    h = jnp.concatenate(hs, axis=1)
    s = hs[0]
    for f in range(1, _F):
        s = s + hs[f]
    sq_of_sum = jnp.sum(s * s, axis=1, keepdims=True)
    sum_of_sq = jnp.sum(h * h, axis=1, keepdims=True)
    cross = 0.5 * (sq_of_sum - sum_of_sq)
    lin = jnp.sum(l16_ref[...] * m16_ref[...], axis=1, keepdims=True)
    lin = lin + bias_ref[0, 0]

    def layer(a, w_ref, b_ref, g_ref, e_ref):
        z = jnp.dot(a, w_ref[...], preferred_element_type=jnp.float32)
        scale = g_ref[...] * inv
        shift = b_ref[...] * scale + e_ref[...]
        return jnp.maximum(z * scale + shift, 0.0)

    a = layer(h, w1_ref, b1_ref, g1_ref, e1_ref)
    a = layer(a, w2_ref, b2_ref, g2_ref, e2_ref)
    a = layer(a, w3_ref, b3_ref, g3_ref, e3_ref)
    mlp = jnp.dot(a, wout_ref[...], preferred_element_type=jnp.float32)
    mlp = mlp + bout_ref[0, 0]
    v = lin + cross + mlp
    out_ref[...] = 1.0 / (1.0 + jnp.exp(-v))


def _tc_mlp(h128, par, l16, m16, bias, W1, b1, g1, be1, W2, b2, g2, be2, W3,
            b3, g3, be3, Wout, bout):
    grid = (_B // _BM,)

    def brow(shape):
        return pl.BlockSpec(shape, lambda i: (i, 0))

    def bfull(shape):
        return pl.BlockSpec(shape, lambda i: (0, 0))

    in_specs = [
        brow((_BM, _F * 128)),
        brow((_BM, _F)),
        brow((_BM, _F * 16)),
        brow((_BM, _F * 16)),
        bfull((1, 1)),
        bfull((_H, 400)), bfull((1, 400)), bfull((1, 400)), bfull((1, 400)),
        bfull((400, 400)), bfull((1, 400)), bfull((1, 400)), bfull((1, 400)),
        bfull((400, 400)), bfull((1, 400)), bfull((1, 400)), bfull((1, 400)),
        bfull((400, 1)), bfull((1, 1)),
    ]
    out_specs = pl.BlockSpec((_BM, 1), lambda i: (i, 0))
    return pl.pallas_call(
        _tc_body,
        grid=grid,
        in_specs=in_specs,
        out_specs=out_specs,
        out_shape=jax.ShapeDtypeStruct((_B, 1), jnp.float32),
    )(h128, par, l16, m16, bias.reshape(1, 1),
      W1, b1.reshape(1, 400), g1.reshape(1, 400), be1.reshape(1, 400),
      W2, b2.reshape(1, 400), g2.reshape(1, 400), be2.reshape(1, 400),
      W3, b3.reshape(1, 400), g3.reshape(1, 400), be3.reshape(1, 400),
      Wout, bout.reshape(1, 1))


def kernel(x, linear_w, embed_w, bias, W1, b1, g1, be1, W2, b2, g2, be2, W3,
           b3, g3, be3, Wout, bout):
    xi = x + jnp.asarray(_OFFSETS, dtype=x.dtype)[None, :]
    xif = xi.reshape(-1)
    q3 = (xif >> 1).reshape(_NW, _CHUNKS_PER_W, _CHUNK)
    ql3 = (xif >> 4).reshape(_NW, _CHUNKS_PER_W, _CHUNK)
    t2 = embed_w[:2 * _V2].reshape(_V2, 128)
    lp = linear_w.reshape(-1)[:16 * _VL].reshape(_VL, 16)
    pairs = _sc_gather_emb(q3, t2)
    lin16 = _sc_gather_lin(ql3, lp)
    h128 = pairs.reshape(_B, _F * 128)
    l16 = lin16.reshape(_B, _F * 16)
    par = (xi & 1).astype(jnp.float32)
    m16 = jax.nn.one_hot(xi & 15, 16, dtype=jnp.float32).reshape(_B, _F * 16)
    return _tc_mlp(h128, par, l16, m16, bias, W1, b1, g1, be1, W2, b2, g2,
                   be2, W3, b3, g3, be3, Wout, bout)


# speedup vs baseline: 1.4226x; 1.4008x over previous
import functools

import jax
import jax.numpy as jnp
import numpy as np
from jax import lax
from jax.experimental import pallas as pl
from jax.experimental.pallas import tpu as pltpu
from jax.experimental.pallas import tpu_sc as plsc

_FEATURE_FIELDS = [100000] * 26
_OFFSETS = np.array((0, *np.cumsum(_FEATURE_FIELDS)[:-1]), dtype=np.int32)
_B = 4096
_F = 26
_D = 64
_BF = _B * _F
_BN_EPS = 1e-3

_NC = 2
_NS = 16
_NW = _NC * _NS
_CH = 32
_BB = 16
_NG = _F // 2
_NBLK = _B // _BB
_NCHUNK = _NBLK * _NG
_CPW = _NCHUNK // _NW

_VE = 2600000
_VL = 162500

_CHUNKS_PER_W = _BF // (_NW * 128)
_N_PER_W = 128 * _CHUNKS_PER_W

_BM = 256
_H = _F * _D


def _sc_gather_emb(xic, t_in):
    mesh = plsc.VectorSubcoreMesh(core_axis_name="c", subcore_axis_name="s")

    @functools.partial(
        pl.kernel,
        mesh=mesh,
        compiler_params=pltpu.CompilerParams(use_tc_tiling_on_sc=True,
                                             needs_layout_passes=False),
        out_type=(
            jax.ShapeDtypeStruct((_B, _H), jnp.float32),
            jax.ShapeDtypeStruct((_CH, 8, _D), jnp.float32),
        ),
        scratch_types=(
            pltpu.VMEM((_CPW, _CH), jnp.int32),
            pltpu.SMEM((2, _CH), jnp.int32),
            pltpu.VMEM((2, _CH, 8, _D), jnp.float32),
            pltpu.VMEM((_BB, 128), jnp.float32),
            pltpu.SemaphoreType.DMA,
            pltpu.SemaphoreType.DMA,
        ),
    )
    def k(xi_hbm, t_hbm, out_hbm, dummy_hbm, xiv, xis, sbuf, obuf, sem0,
          sem1):
        wid = lax.axis_index("s") * _NC + lax.axis_index("c")
        sems = (sem0, sem1)
        lanes = lax.broadcasted_iota(jnp.int32, (16,), 0)
        pltpu.sync_copy(xi_hbm.at[wid], xiv)

        def issue(j, slot):
            for grp in range(_CH // 16):
                vec = xiv[j, pl.ds(grp * 16, 16)]

                def gb(lane, cc):
                    r = jnp.sum(jnp.where(lanes == lane, vec, 0))
                    i = grp * 16 + lane
                    xis[slot, i] = r
                    r8 = pl.multiple_of(r & (-8), 8)
                    pltpu.async_copy(t_hbm.at[pl.ds(r8, 8), :],
                                     sbuf.at[slot].at[i], sems[slot])
                    return cc

                lax.fori_loop(0, 16, gb, 0)

        def drain_ext(j, slot):
            pltpu.make_async_copy(dummy_hbm, sbuf.at[slot],
                                  sems[slot]).wait()

            def ext(i, cc):
                r = xis[slot, i]
                m = r & 7
                row = i >> 1
                col = (i & 1) * _D
                for g in range(_D // 16):
                    obuf[row, pl.ds(col + g * 16, 16)] = (
                        sbuf[slot, i, m, pl.ds(g * 16, 16)])
                return cc

            lax.fori_loop(0, _CH, ext, 0)
            cid = wid * _CPW + j
            gp = lax.rem(cid, _NG)
            b0 = lax.div(cid, _NG) * _BB
            pltpu.sync_copy(
                obuf,
                out_hbm.at[pl.ds(pl.multiple_of(b0, _BB), _BB),
                           pl.ds(pl.multiple_of(gp * 128, 128), 128)])

        issue(0, 0)

        def pair(p, carry):
            j0 = p * 2
            issue(j0 + 1, 1)
            drain_ext(j0, 0)

            @pl.when(j0 + 2 < _CPW)
            def _():
                issue(j0 + 2, 0)

            drain_ext(j0 + 1, 1)
            return carry

        lax.fori_loop(0, _CPW // 2, pair, 0)

    return k(xic, t_in)


def _sc_gather_lin(ql3, lp):
    mesh = plsc.VectorSubcoreMesh(core_axis_name="c", subcore_axis_name="s")

    @functools.partial(
        pl.kernel,
        mesh=mesh,
        compiler_params=pltpu.CompilerParams(use_tc_tiling_on_sc=False),
        out_type=jax.ShapeDtypeStruct((_BF, 16), jnp.float32),
        scratch_types=(
            pltpu.VMEM((_CHUNKS_PER_W, 128), jnp.int32),
            pltpu.VMEM((128, 16), jnp.float32),
            pltpu.SemaphoreType.DMA,
        ),
    )
    def k(ql_hbm, lp_hbm, lout_hbm, qlv, lbuf, lsem):
        wid = lax.axis_index("s") * _NC + lax.axis_index("c")
        base = wid * _N_PER_W
        pltpu.sync_copy(ql_hbm.at[wid], qlv)

        def body(j, carry):
            pltpu.async_copy(lp_hbm.at[qlv.at[j]], lbuf, lsem).wait()
            pltpu.sync_copy(lbuf, lout_hbm.at[pl.ds(base + j * 128, 128)])
            return carry

        lax.fori_loop(0, _CHUNKS_PER_W, body, 0)

    return k(ql3, lp)


def _tc_body(h_ref, l16_ref, m16_ref, bias_ref,
             w1_ref, b1_ref, g1_ref, e1_ref,
             w2_ref, b2_ref, g2_ref, e2_ref,
             w3_ref, b3_ref, g3_ref, e3_ref,
             wout_ref, bout_ref, out_ref):
    inv = lax.rsqrt(jnp.float32(1.0 + _BN_EPS))
    h = h_ref[...]
    s = h[:, 0:_D]
    for f in range(1, _F):
        s = s + h[:, f * _D:(f + 1) * _D]
    sq_of_sum = jnp.sum(s * s, axis=1, keepdims=True)
    sum_of_sq = jnp.sum(h * h, axis=1, keepdims=True)
    cross = 0.5 * (sq_of_sum - sum_of_sq)
    lin = jnp.sum(l16_ref[...] * m16_ref[...], axis=1, keepdims=True)
    lin = lin + bias_ref[0, 0]

    def layer(a, w_ref, b_ref, g_ref, e_ref):
        z = jnp.dot(a, w_ref[...], preferred_element_type=jnp.float32)
        scale = g_ref[...] * inv
        shift = b_ref[...] * scale + e_ref[...]
        return jnp.maximum(z * scale + shift, 0.0)

    a = layer(h, w1_ref, b1_ref, g1_ref, e1_ref)
    a = layer(a, w2_ref, b2_ref, g2_ref, e2_ref)
    a = layer(a, w3_ref, b3_ref, g3_ref, e3_ref)
    mlp = jnp.dot(a, wout_ref[...], preferred_element_type=jnp.float32)
    mlp = mlp + bout_ref[0, 0]
    v = lin + cross + mlp
    out_ref[...] = 1.0 / (1.0 + jnp.exp(-v))


def _tc_mlp(h, l16, m16, bias, W1, b1, g1, be1, W2, b2, g2, be2, W3, b3, g3,
            be3, Wout, bout):
    grid = (_B // _BM,)

    def brow(shape):
        return pl.BlockSpec(shape, lambda i: (i, 0))

    def bfull(shape):
        return pl.BlockSpec(shape, lambda i: (0, 0))

    in_specs = [
        brow((_BM, _H)),
        brow((_BM, _F * 16)),
        brow((_BM, _F * 16)),
        bfull((1, 1)),
        bfull((_H, 400)), bfull((1, 400)), bfull((1, 400)), bfull((1, 400)),
        bfull((400, 400)), bfull((1, 400)), bfull((1, 400)), bfull((1, 400)),
        bfull((400, 400)), bfull((1, 400)), bfull((1, 400)), bfull((1, 400)),
        bfull((400, 1)), bfull((1, 1)),
    ]
    out_specs = pl.BlockSpec((_BM, 1), lambda i: (i, 0))
    return pl.pallas_call(
        _tc_body,
        grid=grid,
        in_specs=in_specs,
        out_specs=out_specs,
        out_shape=jax.ShapeDtypeStruct((_B, 1), jnp.float32),
    )(h, l16, m16, bias.reshape(1, 1),
      W1, b1.reshape(1, 400), g1.reshape(1, 400), be1.reshape(1, 400),
      W2, b2.reshape(1, 400), g2.reshape(1, 400), be2.reshape(1, 400),
      W3, b3.reshape(1, 400), g3.reshape(1, 400), be3.reshape(1, 400),
      Wout, bout.reshape(1, 1))


def kernel(x, linear_w, embed_w, bias, W1, b1, g1, be1, W2, b2, g2, be2, W3,
           b3, g3, be3, Wout, bout):
    xi = x + jnp.asarray(_OFFSETS, dtype=x.dtype)[None, :]
    xic = (xi.reshape(_NBLK, _BB, _NG, 2)
           .transpose(0, 2, 1, 3)
           .reshape(_NW, _CPW, _CH))
    ql3 = (xi.reshape(-1) >> 4).reshape(_NW, _CHUNKS_PER_W, 128)
    t_in = embed_w[:_VE]
    lp = linear_w.reshape(-1)[:16 * _VL].reshape(_VL, 16)
    h, _ = _sc_gather_emb(xic, t_in)
    lin16 = _sc_gather_lin(ql3, lp)
    l16 = lin16.reshape(_B, _F * 16)
    m16 = jax.nn.one_hot(xi & 15, 16, dtype=jnp.float32).reshape(_B, _F * 16)
    return _tc_mlp(h, l16, m16, bias, W1, b1, g1, be1, W2, b2, g2, be2, W3,
                   b3, g3, be3, Wout, bout)


# speedup vs baseline: 1.4236x; 1.0007x over previous
import functools

import jax
import jax.numpy as jnp
import numpy as np
from jax import lax
from jax.experimental import pallas as pl
from jax.experimental.pallas import tpu as pltpu
from jax.experimental.pallas import tpu_sc as plsc

_FEATURE_FIELDS = [100000] * 26
_OFFSETS = np.array((0, *np.cumsum(_FEATURE_FIELDS)[:-1]), dtype=np.int32)
_B = 4096
_F = 26
_D = 64
_BF = _B * _F
_BN_EPS = 1e-3

_NC = 2
_NS = 16
_NW = _NC * _NS
_CH = 32
_BB = 16
_NG = _F // 2
_NBLK = _B // _BB
_NCHUNK = _NBLK * _NG
_CPW = _NCHUNK // _NW

_VE = 2600000
_VL = 162500

_CHUNKS_PER_W = _BF // (_NW * 128)
_N_PER_W = 128 * _CHUNKS_PER_W

_BM = 256
_H = _F * _D


def _sc_gather_emb(xic, t_in):
    mesh = plsc.VectorSubcoreMesh(core_axis_name="c", subcore_axis_name="s")

    @functools.partial(
        pl.kernel,
        mesh=mesh,
        compiler_params=pltpu.CompilerParams(use_tc_tiling_on_sc=True,
                                             needs_layout_passes=False),
        out_type=(
            jax.ShapeDtypeStruct((_B, _H), jnp.float32),
            jax.ShapeDtypeStruct((_CH, 8, _D), jnp.float32),
        ),
        scratch_types=(
            pltpu.VMEM((_CPW, _CH), jnp.int32),
            pltpu.SMEM((2, _CH), jnp.int32),
            pltpu.VMEM((2, _CH, 8, _D), jnp.float32),
            pltpu.VMEM((_BB, 128), jnp.float32),
            pltpu.SemaphoreType.DMA,
            pltpu.SemaphoreType.DMA,
        ),
    )
    def k(xi_hbm, t_hbm, out_hbm, dummy_hbm, xiv, xis, sbuf, obuf, sem0,
          sem1):
        wid = lax.axis_index("s") * _NC + lax.axis_index("c")
        sems = (sem0, sem1)
        lanes = lax.broadcasted_iota(jnp.int32, (16,), 0)
        pltpu.sync_copy(xi_hbm.at[wid], xiv)

        def issue(j, slot):
            for grp in range(_CH // 16):
                vec = xiv[j, pl.ds(grp * 16, 16)]

                def gb(lane, cc):
                    r = jnp.sum(jnp.where(lanes == lane, vec, 0))
                    i = grp * 16 + lane
                    xis[slot, i] = r
                    r8 = pl.multiple_of(r & (-8), 8)
                    pltpu.async_copy(t_hbm.at[pl.ds(r8, 8), :],
                                     sbuf.at[slot].at[i], sems[slot])
                    return cc

                lax.fori_loop(0, 16, gb, 0)

        def drain_ext(j, slot):
            pltpu.make_async_copy(dummy_hbm, sbuf.at[slot],
                                  sems[slot]).wait()

            def ext(i, cc):
                r = xis[slot, i]
                m = r & 7
                row = i >> 1
                col = (i & 1) * _D
                for g in range(_D // 16):
                    obuf[row, pl.ds(col + g * 16, 16)] = (
                        sbuf[slot, i, m, pl.ds(g * 16, 16)])
                return cc

            lax.fori_loop(0, _CH, ext, 0)
            cid = wid * _CPW + j
            gp = lax.rem(cid, _NG)
            b0 = lax.div(cid, _NG) * _BB
            pltpu.sync_copy(
                obuf,
                out_hbm.at[pl.ds(pl.multiple_of(b0, _BB), _BB),
                           pl.ds(pl.multiple_of(gp * 128, 128), 128)])

        issue(0, 0)

        def pair(p, carry):
            j0 = p * 2
            issue(j0 + 1, 1)
            drain_ext(j0, 0)

            @pl.when(j0 + 2 < _CPW)
            def _():
                issue(j0 + 2, 0)

            drain_ext(j0 + 1, 1)
            return carry

        lax.fori_loop(0, _CPW // 2, pair, 0)

    return k(xic, t_in)


def _sc_gather_lin(ql3, lp):
    mesh = plsc.VectorSubcoreMesh(core_axis_name="c", subcore_axis_name="s")

    @functools.partial(
        pl.kernel,
        mesh=mesh,
        compiler_params=pltpu.CompilerParams(use_tc_tiling_on_sc=False),
        out_type=jax.ShapeDtypeStruct((_BF, 16), jnp.float32),
        scratch_types=(
            pltpu.VMEM((_CHUNKS_PER_W, 128), jnp.int32),
            pltpu.VMEM((128, 16), jnp.float32),
            pltpu.SemaphoreType.DMA,
        ),
    )
    def k(ql_hbm, lp_hbm, lout_hbm, qlv, lbuf, lsem):
        wid = lax.axis_index("s") * _NC + lax.axis_index("c")
        base = wid * _N_PER_W
        pltpu.sync_copy(ql_hbm.at[wid], qlv)

        def body(j, carry):
            pltpu.async_copy(lp_hbm.at[qlv.at[j]], lbuf, lsem).wait()
            pltpu.sync_copy(lbuf, lout_hbm.at[pl.ds(base + j * 128, 128)])
            return carry

        lax.fori_loop(0, _CHUNKS_PER_W, body, 0)

    return k(ql3, lp)


def _tc_body(h_ref, l16_ref, m16_ref, bias_ref,
             w1_ref, b1_ref, g1_ref, e1_ref,
             w2_ref, b2_ref, g2_ref, e2_ref,
             w3_ref, b3_ref, g3_ref, e3_ref,
             wout_ref, bout_ref, out_ref):
    inv = lax.rsqrt(jnp.float32(1.0 + _BN_EPS))
    h = h_ref[...]
    s = h[:, 0:_D]
    for f in range(1, _F):
        s = s + h[:, f * _D:(f + 1) * _D]
    sq_of_sum = jnp.sum(s * s, axis=1, keepdims=True)
    sum_of_sq = jnp.sum(h * h, axis=1, keepdims=True)
    cross = 0.5 * (sq_of_sum - sum_of_sq)
    lin = jnp.sum(l16_ref[...] * m16_ref[...], axis=1, keepdims=True)
    lin = lin + bias_ref[0, 0]

    def layer(a, w_ref, b_ref, g_ref, e_ref):
        z = jnp.dot(a, w_ref[...], preferred_element_type=jnp.float32)
        scale = g_ref[...] * inv
        shift = b_ref[...] * scale + e_ref[...]
        return jnp.maximum(z * scale + shift, 0.0)

    a = layer(h, w1_ref, b1_ref, g1_ref, e1_ref)
    a = layer(a, w2_ref, b2_ref, g2_ref, e2_ref)
    a = layer(a, w3_ref, b3_ref, g3_ref, e3_ref)
    mlp = jnp.dot(a, wout_ref[...], preferred_element_type=jnp.float32)
    mlp = mlp + bout_ref[0, 0]
    v = lin + cross + mlp
    out_ref[...] = 1.0 / (1.0 + jnp.exp(-v))


def _tc_mlp(h, l16, m16, bias, W1, b1, g1, be1, W2, b2, g2, be2, W3, b3, g3,
            be3, Wout, bout):
    grid = (_B // _BM,)

    def brow(shape):
        return pl.BlockSpec(shape, lambda i: (i, 0))

    def bfull(shape):
        return pl.BlockSpec(shape, lambda i: (0, 0))

    in_specs = [
        brow((_BM, _H)),
        brow((_BM, _F * 16)),
        brow((_BM, _F * 16)),
        bfull((1, 1)),
        bfull((_H, 400)), bfull((1, 400)), bfull((1, 400)), bfull((1, 400)),
        bfull((400, 400)), bfull((1, 400)), bfull((1, 400)), bfull((1, 400)),
        bfull((400, 400)), bfull((1, 400)), bfull((1, 400)), bfull((1, 400)),
        bfull((400, 1)), bfull((1, 1)),
    ]
    out_specs = pl.BlockSpec((_BM, 1), lambda i: (i, 0))
    return pl.pallas_call(
        _tc_body,
        grid=grid,
        in_specs=in_specs,
        out_specs=out_specs,
        out_shape=jax.ShapeDtypeStruct((_B, 1), jnp.float32),
    )(h, l16, m16, bias.reshape(1, 1),
      W1, b1.reshape(1, 400), g1.reshape(1, 400), be1.reshape(1, 400),
      W2, b2.reshape(1, 400), g2.reshape(1, 400), be2.reshape(1, 400),
      W3, b3.reshape(1, 400), g3.reshape(1, 400), be3.reshape(1, 400),
      Wout, bout.reshape(1, 1))


def kernel(x, linear_w, embed_w, bias, W1, b1, g1, be1, W2, b2, g2, be2, W3,
           b3, g3, be3, Wout, bout):
    xi = x + jnp.asarray(_OFFSETS, dtype=x.dtype)[None, :]
    xic = (xi.reshape(_NBLK, _BB, _NG, 2)
           .transpose(0, 2, 1, 3)
           .reshape(_NW, _CPW, _CH))
    ql3 = (xi.reshape(-1) >> 4).reshape(_NW, _CHUNKS_PER_W, 128)
    t_in = embed_w
    lp = linear_w.reshape(-1)[:16 * _VL].reshape(_VL, 16)
    h, _ = _sc_gather_emb(xic, t_in)
    lin16 = _sc_gather_lin(ql3, lp)
    l16 = lin16.reshape(_B, _F * 16)
    m16 = jax.nn.one_hot(xi & 15, 16, dtype=jnp.float32).reshape(_B, _F * 16)
    return _tc_mlp(h, l16, m16, bias, W1, b1, g1, be1, W2, b2, g2, be2, W3,
                   b3, g3, be3, Wout, bout)


# speedup vs baseline: 2.1093x; 1.4817x over previous
import functools

import jax
import jax.numpy as jnp
import numpy as np
from jax import lax
from jax.experimental import pallas as pl
from jax.experimental.pallas import tpu as pltpu
from jax.experimental.pallas import tpu_sc as plsc

_FEATURE_FIELDS = [100000] * 26
_OFFSETS = np.array((0, *np.cumsum(_FEATURE_FIELDS)[:-1]), dtype=np.int32)
_B = 4096
_F = 26
_D = 64
_BF = _B * _F
_BN_EPS = 1e-3

_NC = 2
_NS = 16
_NW = _NC * _NS
_CH = 32
_BB = 16
_NG = _F // 2
_NBLK = _B // _BB
_NCHUNK = _NBLK * _NG
_CPW = _NCHUNK // _NW

_VE = 2600000
_VL = 162500

_CHUNKS_PER_W = _BF // (_NW * 128)
_N_PER_W = 128 * _CHUNKS_PER_W

_BM = 256
_H = _F * _D


def _sc_gather_emb(xic, t_in):
    mesh = plsc.VectorSubcoreMesh(core_axis_name="c", subcore_axis_name="s")

    @functools.partial(
        pl.kernel,
        mesh=mesh,
        compiler_params=pltpu.CompilerParams(use_tc_tiling_on_sc=True,
                                             needs_layout_passes=False),
        out_type=(
            jax.ShapeDtypeStruct((_B, _H), jnp.float32),
            jax.ShapeDtypeStruct((_CH, 8, _D), jnp.float32),
        ),
        scratch_types=(
            pltpu.VMEM((_CPW, _CH), jnp.int32),
            pltpu.SMEM((2, _CH), jnp.int32),
            pltpu.VMEM((2, _CH, 8, _D), jnp.float32),
            pltpu.VMEM((_BB, 128), jnp.float32),
            pltpu.SemaphoreType.DMA,
            pltpu.SemaphoreType.DMA,
        ),
    )
    def k(xi_hbm, t_hbm, out_hbm, dummy_hbm, xiv, xis, sbuf, obuf, sem0,
          sem1):
        wid = lax.axis_index("s") * _NC + lax.axis_index("c")
        sems = (sem0, sem1)
        lanes = lax.broadcasted_iota(jnp.int32, (16,), 0)
        pltpu.sync_copy(xi_hbm.at[wid], xiv)

        def issue(j, slot):
            for grp in range(_CH // 16):
                vec = xiv[j, pl.ds(grp * 16, 16)]

                def gb(lane, cc):
                    r = jnp.sum(jnp.where(lanes == lane, vec, 0))
                    i = grp * 16 + lane
                    xis[slot, i] = r
                    q = lax.shift_right_logical(r, 3)
                    pltpu.async_copy(t_hbm.at[q],
                                     sbuf.at[slot].at[i], sems[slot])
                    return cc

                lax.fori_loop(0, 16, gb, 0)

        def drain_ext(j, slot):
            pltpu.make_async_copy(dummy_hbm, sbuf.at[slot],
                                  sems[slot]).wait()

            def ext(i, cc):
                r = xis[slot, i]
                m = r & 7
                row = i >> 1
                col = (i & 1) * _D
                for g in range(_D // 16):
                    obuf[row, pl.ds(col + g * 16, 16)] = (
                        sbuf[slot, i, m, pl.ds(g * 16, 16)])
                return cc

            lax.fori_loop(0, _CH, ext, 0)
            cid = wid * _CPW + j
            gp = lax.rem(cid, _NG)
            b0 = lax.div(cid, _NG) * _BB
            pltpu.sync_copy(
                obuf,
                out_hbm.at[pl.ds(pl.multiple_of(b0, _BB), _BB),
                           pl.ds(pl.multiple_of(gp * 128, 128), 128)])

        issue(0, 0)

        def pair(p, carry):
            j0 = p * 2
            issue(j0 + 1, 1)
            drain_ext(j0, 0)

            @pl.when(j0 + 2 < _CPW)
            def _():
                issue(j0 + 2, 0)

            drain_ext(j0 + 1, 1)
            return carry

        lax.fori_loop(0, _CPW // 2, pair, 0)

    return k(xic, t_in)


def _sc_gather_lin(ql3, lp):
    mesh = plsc.VectorSubcoreMesh(core_axis_name="c", subcore_axis_name="s")

    @functools.partial(
        pl.kernel,
        mesh=mesh,
        compiler_params=pltpu.CompilerParams(use_tc_tiling_on_sc=False),
        out_type=jax.ShapeDtypeStruct((_BF, 16), jnp.float32),
        scratch_types=(
            pltpu.VMEM((_CHUNKS_PER_W, 128), jnp.int32),
            pltpu.VMEM((128, 16), jnp.float32),
            pltpu.SemaphoreType.DMA,
        ),
    )
    def k(ql_hbm, lp_hbm, lout_hbm, qlv, lbuf, lsem):
        wid = lax.axis_index("s") * _NC + lax.axis_index("c")
        base = wid * _N_PER_W
        pltpu.sync_copy(ql_hbm.at[wid], qlv)

        def body(j, carry):
            pltpu.async_copy(lp_hbm.at[qlv.at[j]], lbuf, lsem).wait()
            pltpu.sync_copy(lbuf, lout_hbm.at[pl.ds(base + j * 128, 128)])
            return carry

        lax.fori_loop(0, _CHUNKS_PER_W, body, 0)

    return k(ql3, lp)


def _tc_body(h_ref, l16_ref, m16_ref, bias_ref,
             w1_ref, b1_ref, g1_ref, e1_ref,
             w2_ref, b2_ref, g2_ref, e2_ref,
             w3_ref, b3_ref, g3_ref, e3_ref,
             wout_ref, bout_ref, out_ref):
    inv = lax.rsqrt(jnp.float32(1.0 + _BN_EPS))
    h = h_ref[...]
    s = h[:, 0:_D]
    for f in range(1, _F):
        s = s + h[:, f * _D:(f + 1) * _D]
    sq_of_sum = jnp.sum(s * s, axis=1, keepdims=True)
    sum_of_sq = jnp.sum(h * h, axis=1, keepdims=True)
    cross = 0.5 * (sq_of_sum - sum_of_sq)
    lin = jnp.sum(l16_ref[...] * m16_ref[...], axis=1, keepdims=True)
    lin = lin + bias_ref[0, 0]

    def layer(a, w_ref, b_ref, g_ref, e_ref):
        z = jnp.dot(a, w_ref[...], preferred_element_type=jnp.float32)
        scale = g_ref[...] * inv
        shift = b_ref[...] * scale + e_ref[...]
        return jnp.maximum(z * scale + shift, 0.0)

    a = layer(h, w1_ref, b1_ref, g1_ref, e1_ref)
    a = layer(a, w2_ref, b2_ref, g2_ref, e2_ref)
    a = layer(a, w3_ref, b3_ref, g3_ref, e3_ref)
    mlp = jnp.dot(a, wout_ref[...], preferred_element_type=jnp.float32)
    mlp = mlp + bout_ref[0, 0]
    v = lin + cross + mlp
    out_ref[...] = 1.0 / (1.0 + jnp.exp(-v))


def _tc_mlp(h, l16, m16, bias, W1, b1, g1, be1, W2, b2, g2, be2, W3, b3, g3,
            be3, Wout, bout):
    grid = (_B // _BM,)

    def brow(shape):
        return pl.BlockSpec(shape, lambda i: (i, 0))

    def bfull(shape):
        return pl.BlockSpec(shape, lambda i: (0, 0))

    in_specs = [
        brow((_BM, _H)),
        brow((_BM, _F * 16)),
        brow((_BM, _F * 16)),
        bfull((1, 1)),
        bfull((_H, 400)), bfull((1, 400)), bfull((1, 400)), bfull((1, 400)),
        bfull((400, 400)), bfull((1, 400)), bfull((1, 400)), bfull((1, 400)),
        bfull((400, 400)), bfull((1, 400)), bfull((1, 400)), bfull((1, 400)),
        bfull((400, 1)), bfull((1, 1)),
    ]
    out_specs = pl.BlockSpec((_BM, 1), lambda i: (i, 0))
    return pl.pallas_call(
        _tc_body,
        grid=grid,
        in_specs=in_specs,
        out_specs=out_specs,
        out_shape=jax.ShapeDtypeStruct((_B, 1), jnp.float32),
    )(h, l16, m16, bias.reshape(1, 1),
      W1, b1.reshape(1, 400), g1.reshape(1, 400), be1.reshape(1, 400),
      W2, b2.reshape(1, 400), g2.reshape(1, 400), be2.reshape(1, 400),
      W3, b3.reshape(1, 400), g3.reshape(1, 400), be3.reshape(1, 400),
      Wout, bout.reshape(1, 1))


def kernel(x, linear_w, embed_w, bias, W1, b1, g1, be1, W2, b2, g2, be2, W3,
           b3, g3, be3, Wout, bout):
    xi = x + jnp.asarray(_OFFSETS, dtype=x.dtype)[None, :]
    xic = (xi.reshape(_NBLK, _BB, _NG, 2)
           .transpose(0, 2, 1, 3)
           .reshape(_NW, _CPW, _CH))
    ql3 = (xi.reshape(-1) >> 4).reshape(_NW, _CHUNKS_PER_W, 128)
    t_in = embed_w[:_VE].reshape(_VE // 8, 8, _D)
    lp = linear_w.reshape(-1)[:16 * _VL].reshape(_VL, 16)
    h, _ = _sc_gather_emb(xic, t_in)
    lin16 = _sc_gather_lin(ql3, lp)
    l16 = lin16.reshape(_B, _F * 16)
    m16 = jax.nn.one_hot(xi & 15, 16, dtype=jnp.float32).reshape(_B, _F * 16)
    return _tc_mlp(h, l16, m16, bias, W1, b1, g1, be1, W2, b2, g2, be2, W3,
                   b3, g3, be3, Wout, bout)


# speedup vs baseline: 2.1268x; 1.0083x over previous
import functools

import jax
import jax.numpy as jnp
import numpy as np
from jax import lax
from jax.experimental import pallas as pl
from jax.experimental.pallas import tpu as pltpu
from jax.experimental.pallas import tpu_sc as plsc

_FEATURE_FIELDS = [100000] * 26
_OFFSETS = np.array((0, *np.cumsum(_FEATURE_FIELDS)[:-1]), dtype=np.int32)
_B = 4096
_F = 26
_D = 64
_BF = _B * _F
_BN_EPS = 1e-3

_NC = 2
_NS = 16
_NW = _NC * _NS
_CH = 32
_BB = 16
_NG = _F // 2
_NBLK = _B // _BB
_NCHUNK = _NBLK * _NG
_CPW = _NCHUNK // _NW

_VE = 2600000
_VL = 162500

_CHUNKS_PER_W = _BF // (_NW * 128)
_N_PER_W = 128 * _CHUNKS_PER_W

_BM = 512
_H = _F * _D


def _sc_gather_emb(xic, t_in):
    mesh = plsc.VectorSubcoreMesh(core_axis_name="c", subcore_axis_name="s")

    @functools.partial(
        pl.kernel,
        mesh=mesh,
        compiler_params=pltpu.CompilerParams(use_tc_tiling_on_sc=True,
                                             needs_layout_passes=False),
        out_type=(
            jax.ShapeDtypeStruct((_B, _H), jnp.float32),
            jax.ShapeDtypeStruct((_CH, 8, _D), jnp.float32),
        ),
        scratch_types=(
            pltpu.VMEM((_CPW, _CH), jnp.int32),
            pltpu.SMEM((2, _CH), jnp.int32),
            pltpu.VMEM((2, _CH, 8, _D), jnp.float32),
            pltpu.VMEM((_BB, 128), jnp.float32),
            pltpu.SemaphoreType.DMA,
            pltpu.SemaphoreType.DMA,
        ),
    )
    def k(xi_hbm, t_hbm, out_hbm, dummy_hbm, xiv, xis, sbuf, obuf, sem0,
          sem1):
        wid = lax.axis_index("s") * _NC + lax.axis_index("c")
        sems = (sem0, sem1)
        lanes = lax.broadcasted_iota(jnp.int32, (16,), 0)
        pltpu.sync_copy(xi_hbm.at[wid], xiv)

        def issue(j, slot):
            for grp in range(_CH // 16):
                vec = xiv[j, pl.ds(grp * 16, 16)]

                def gb(lane, cc):
                    r = jnp.sum(jnp.where(lanes == lane, vec, 0))
                    i = grp * 16 + lane
                    xis[slot, i] = r
                    q = lax.shift_right_logical(r, 3)
                    pltpu.async_copy(t_hbm.at[q],
                                     sbuf.at[slot].at[i], sems[slot])
                    return cc

                lax.fori_loop(0, 16, gb, 0)

        def drain_ext(j, slot):
            pltpu.make_async_copy(dummy_hbm, sbuf.at[slot],
                                  sems[slot]).wait()

            def ext(i, cc):
                r = xis[slot, i]
                m = r & 7
                row = i >> 1
                col = (i & 1) * _D
                for g in range(_D // 16):
                    obuf[row, pl.ds(col + g * 16, 16)] = (
                        sbuf[slot, i, m, pl.ds(g * 16, 16)])
                return cc

            lax.fori_loop(0, _CH, ext, 0)
            cid = wid * _CPW + j
            gp = lax.rem(cid, _NG)
            b0 = lax.div(cid, _NG) * _BB
            pltpu.sync_copy(
                obuf,
                out_hbm.at[pl.ds(pl.multiple_of(b0, _BB), _BB),
                           pl.ds(pl.multiple_of(gp * 128, 128), 128)])

        issue(0, 0)

        def pair(p, carry):
            j0 = p * 2
            issue(j0 + 1, 1)
            drain_ext(j0, 0)

            @pl.when(j0 + 2 < _CPW)
            def _():
                issue(j0 + 2, 0)

            drain_ext(j0 + 1, 1)
            return carry

        lax.fori_loop(0, _CPW // 2, pair, 0)

    return k(xic, t_in)


def _sc_gather_lin(ql3, lp):
    mesh = plsc.VectorSubcoreMesh(core_axis_name="c", subcore_axis_name="s")

    @functools.partial(
        pl.kernel,
        mesh=mesh,
        compiler_params=pltpu.CompilerParams(use_tc_tiling_on_sc=False),
        out_type=jax.ShapeDtypeStruct((_BF, 16), jnp.float32),
        scratch_types=(
            pltpu.VMEM((_CHUNKS_PER_W, 128), jnp.int32),
            pltpu.VMEM((128, 16), jnp.float32),
            pltpu.SemaphoreType.DMA,
        ),
    )
    def k(ql_hbm, lp_hbm, lout_hbm, qlv, lbuf, lsem):
        wid = lax.axis_index("s") * _NC + lax.axis_index("c")
        base = wid * _N_PER_W
        pltpu.sync_copy(ql_hbm.at[wid], qlv)

        def body(j, carry):
            pltpu.async_copy(lp_hbm.at[qlv.at[j]], lbuf, lsem).wait()
            pltpu.sync_copy(lbuf, lout_hbm.at[pl.ds(base + j * 128, 128)])
            return carry

        lax.fori_loop(0, _CHUNKS_PER_W, body, 0)

    return k(ql3, lp)


def _tc_body(h_ref, l16_ref, m16_ref, bias_ref,
             w1_ref, b1_ref, g1_ref, e1_ref,
             w2_ref, b2_ref, g2_ref, e2_ref,
             w3_ref, b3_ref, g3_ref, e3_ref,
             wout_ref, bout_ref, out_ref):
    inv = lax.rsqrt(jnp.float32(1.0 + _BN_EPS))
    h = h_ref[...]
    s = h[:, 0:_D]
    for f in range(1, _F):
        s = s + h[:, f * _D:(f + 1) * _D]
    sq_of_sum = jnp.sum(s * s, axis=1, keepdims=True)
    sum_of_sq = jnp.sum(h * h, axis=1, keepdims=True)
    cross = 0.5 * (sq_of_sum - sum_of_sq)
    lin = jnp.sum(l16_ref[...] * m16_ref[...], axis=1, keepdims=True)
    lin = lin + bias_ref[0, 0]

    def layer(a, w_ref, b_ref, g_ref, e_ref):
        z = jnp.dot(a, w_ref[...], preferred_element_type=jnp.float32)
        scale = g_ref[...] * inv
        shift = b_ref[...] * scale + e_ref[...]
        return jnp.maximum(z * scale + shift, 0.0)

    a = layer(h, w1_ref, b1_ref, g1_ref, e1_ref)
    a = layer(a, w2_ref, b2_ref, g2_ref, e2_ref)
    a = layer(a, w3_ref, b3_ref, g3_ref, e3_ref)
    mlp = jnp.dot(a, wout_ref[...], preferred_element_type=jnp.float32)
    mlp = mlp + bout_ref[0, 0]
    v = lin + cross + mlp
    out_ref[...] = 1.0 / (1.0 + jnp.exp(-v))


def _tc_mlp(h, l16, m16, bias, W1, b1, g1, be1, W2, b2, g2, be2, W3, b3, g3,
            be3, Wout, bout):
    grid = (_B // _BM,)

    def brow(shape):
        return pl.BlockSpec(shape, lambda i: (i, 0))

    def bfull(shape):
        return pl.BlockSpec(shape, lambda i: (0, 0))

    in_specs = [
        brow((_BM, _H)),
        brow((_BM, _F * 16)),
        brow((_BM, _F * 16)),
        bfull((1, 1)),
        bfull((_H, 400)), bfull((1, 400)), bfull((1, 400)), bfull((1, 400)),
        bfull((400, 400)), bfull((1, 400)), bfull((1, 400)), bfull((1, 400)),
        bfull((400, 400)), bfull((1, 400)), bfull((1, 400)), bfull((1, 400)),
        bfull((400, 1)), bfull((1, 1)),
    ]
    out_specs = pl.BlockSpec((_BM, 1), lambda i: (i, 0))
    return pl.pallas_call(
        _tc_body,
        grid=grid,
        in_specs=in_specs,
        out_specs=out_specs,
        out_shape=jax.ShapeDtypeStruct((_B, 1), jnp.float32),
    )(h, l16, m16, bias.reshape(1, 1),
      W1, b1.reshape(1, 400), g1.reshape(1, 400), be1.reshape(1, 400),
      W2, b2.reshape(1, 400), g2.reshape(1, 400), be2.reshape(1, 400),
      W3, b3.reshape(1, 400), g3.reshape(1, 400), be3.reshape(1, 400),
      Wout, bout.reshape(1, 1))


def kernel(x, linear_w, embed_w, bias, W1, b1, g1, be1, W2, b2, g2, be2, W3,
           b3, g3, be3, Wout, bout):
    xi = x + jnp.asarray(_OFFSETS, dtype=x.dtype)[None, :]
    xic = (xi.reshape(_NBLK, _BB, _NG, 2)
           .transpose(0, 2, 1, 3)
           .reshape(_NW, _CPW, _CH))
    ql3 = (xi.reshape(-1) >> 4).reshape(_NW, _CHUNKS_PER_W, 128)
    t_in = embed_w[:_VE].reshape(_VE // 8, 8, _D)
    lp = linear_w.reshape(-1)[:16 * _VL].reshape(_VL, 16)
    h, _ = _sc_gather_emb(xic, t_in)
    lin16 = _sc_gather_lin(ql3, lp)
    l16 = lin16.reshape(_B, _F * 16)
    m16 = jax.nn.one_hot(xi & 15, 16, dtype=jnp.float32).reshape(_B, _F * 16)
    return _tc_mlp(h, l16, m16, bias, W1, b1, g1, be1, W2, b2, g2, be2, W3,
                   b3, g3, be3, Wout, bout)
